# baseline (speedup 1.0000x reference)
import jax
import jax.numpy as jnp
from jax import lax
from jax.experimental import pallas as pl
from jax.experimental.pallas import tpu as pltpu
from jax.experimental.pallas import tpu_sc as plsc

NUM_COLLISIONS = 4
EMBED_DIM = 64
BATCH = 16384
BAG = 20

NC = 2
NS = 16
NW = NC * NS
LANES = 16

BAGS_PER_W = BATCH // NW
CHUNK = 32
NCHUNK = BAGS_PER_W // CHUNK
ROWS_PER_CHUNK = CHUNK * BAG
IDX_GROUPS = ROWS_PER_CHUNK // 128
IDX_ROWS_W = BAGS_PER_W * BAG // 128
DSLICES = EMBED_DIM // LANES


REPACK_BLKC = 1024


def _repack_body(in_ref, o_ref):
    x = in_ref[...]
    y = x.reshape(64, REPACK_BLKC // 2, 2)
    z = jnp.transpose(y, (1, 2, 0))
    o_ref[...] = z.reshape(REPACK_BLKC // 2, 128)


def _er_body(idx_ref, wr_ref, o_ref):
    r = idx_ref[...] & 3
    wr = wr_ref[...]
    acc = jnp.zeros((idx_ref.shape[0], EMBED_DIM), jnp.float32)
    for k in range(NUM_COLLISIONS):
        cnt = jnp.sum((r == k).astype(jnp.float32), axis=1, keepdims=True)
        acc = acc + cnt * wr[k:k + 1, :]
    o_ref[...] = acc * (1.0 / (BAG * BAG))


def _sc_body(idx_hbm, wq_hbm, er_hbm, out_hbm,
             idxp_v, qflat, rows_a, rows_b, er_a, er_b, out_a, out_b,
             sem_a, sem_b, sem_oa, sem_ob):
    wid = lax.axis_index("s") * NC + lax.axis_index("c")
    bag_base = wid * BAGS_PER_W

    QT = BAGS_PER_W // 4
    for qt in range(4):
        pltpu.sync_copy(idx_hbm.at[pl.ds(bag_base + qt * QT, QT)], idxp_v)

        def compact_body(b, _, qt=qt):
            lo = jnp.right_shift(idxp_v[b, pl.ds(0, LANES)], 2)
            hi = jnp.right_shift(idxp_v[b, pl.ds(LANES, LANES)], 2)
            off = (qt * QT + b) * BAG
            qflat[pl.ds(off, LANES)] = lo
            qflat[pl.ds(off + LANES, LANES)] = hi
            return 0
        lax.fori_loop(0, QT, compact_body, 0)

    def fire(c, rows_v, er_v, sem):
        for k in range(IDX_GROUPS):
            pltpu.async_copy(
                wq_hbm.at[qflat.at[pl.ds(c * ROWS_PER_CHUNK + k * 128, 128)]],
                rows_v.at[pl.ds(k * 128, 128)], sem)
        pltpu.async_copy(er_hbm.at[pl.ds(bag_base + c * CHUNK, CHUNK)],
                         er_v, sem)

    def wait_set(rows_v, er_v, sem):
        pltpu.make_async_copy(wq_hbm.at[pl.ds(0, ROWS_PER_CHUNK)],
                              rows_v, sem).wait()
        pltpu.make_async_copy(er_hbm.at[pl.ds(0, CHUNK)], er_v, sem).wait()

    def accum(c, rows_v, er_v, out_v, sem_o, guard):
        @pl.when(guard)
        def _():
            pltpu.make_async_copy(out_v, out_hbm.at[pl.ds(0, CHUNK)],
                                  sem_o).wait()

        def bag_body(b, _):
            rbase = b * BAG
            accs = [jnp.zeros((LANES,), jnp.float32) for _ in range(DSLICES)]
            for j in range(BAG):
                for s in range(DSLICES):
                    accs[s] = accs[s] + rows_v[rbase + j,
                                               pl.ds(s * LANES, LANES)]
            for s in range(DSLICES):
                out_v[b, pl.ds(s * LANES, LANES)] = (
                    accs[s] * er_v[b, pl.ds(s * LANES, LANES)])
            return 0

        lax.fori_loop(0, CHUNK, bag_body, 0)
        pltpu.async_copy(out_v, out_hbm.at[pl.ds(bag_base + c * CHUNK, CHUNK)],
                         sem_o)

    fire(0, rows_a, er_a, sem_a)

    def pair_body(p, _):
        c0 = 2 * p
        fire(c0 + 1, rows_b, er_b, sem_b)
        wait_set(rows_a, er_a, sem_a)
        accum(c0, rows_a, er_a, out_a, sem_oa, p > 0)

        @pl.when(p < NCHUNK // 2 - 1)
        def _():
            fire(c0 + 2, rows_a, er_a, sem_a)

        wait_set(rows_b, er_b, sem_b)
        accum(c0 + 1, rows_b, er_b, out_b, sem_ob, p > 0)
        return 0

    lax.fori_loop(0, NCHUNK // 2, pair_body, 0)
    pltpu.make_async_copy(out_a, out_hbm.at[pl.ds(0, CHUNK)], sem_oa).wait()
    pltpu.make_async_copy(out_b, out_hbm.at[pl.ds(0, CHUNK)], sem_ob).wait()


_sc_call = pl.kernel(
    _sc_body,
    out_type=jax.ShapeDtypeStruct((BATCH, EMBED_DIM), jnp.float32),
    mesh=plsc.VectorSubcoreMesh(core_axis_name="c", subcore_axis_name="s"),
    scratch_types=[
        pltpu.VMEM((BAGS_PER_W // 4, 128), jnp.int32),
        pltpu.VMEM((BAGS_PER_W * BAG + LANES, ), jnp.int32),
        pltpu.VMEM((ROWS_PER_CHUNK, EMBED_DIM), jnp.float32),
        pltpu.VMEM((ROWS_PER_CHUNK, EMBED_DIM), jnp.float32),
        pltpu.VMEM((CHUNK, EMBED_DIM), jnp.float32),
        pltpu.VMEM((CHUNK, EMBED_DIM), jnp.float32),
        pltpu.VMEM((CHUNK, EMBED_DIM), jnp.float32),
        pltpu.VMEM((CHUNK, EMBED_DIM), jnp.float32),
        pltpu.SemaphoreType.DMA,
        pltpu.SemaphoreType.DMA,
        pltpu.SemaphoreType.DMA,
        pltpu.SemaphoreType.DMA,
    ],
    compiler_params=pltpu.CompilerParams(use_tc_tiling_on_sc=False),
)


def kernel(input, weight_q, weight_r):
    idx = input.astype(jnp.int32)
    blk = 2048
    er = pl.pallas_call(
        _er_body,
        grid=(BATCH // blk,),
        in_specs=[
            pl.BlockSpec((blk, BAG), lambda i: (i, 0)),
            pl.BlockSpec((NUM_COLLISIONS, EMBED_DIM), lambda i: (0, 0)),
        ],
        out_specs=pl.BlockSpec((blk, EMBED_DIM), lambda i: (i, 0)),
        out_shape=jax.ShapeDtypeStruct((BATCH, EMBED_DIM), jnp.float32),
    )(idx, weight_r)
    idxp = jnp.pad(idx, ((0, 0), (0, 128 - BAG)))
    nq = weight_q.shape[0]
    tbl = pl.pallas_call(
        _repack_body,
        grid=(pl.cdiv(nq, REPACK_BLKC),),
        in_specs=[pl.BlockSpec((EMBED_DIM, REPACK_BLKC), lambda i: (0, i))],
        out_specs=pl.BlockSpec((REPACK_BLKC // 2, 128), lambda i: (i, 0)),
        out_shape=jax.ShapeDtypeStruct((nq // 2, 128), jnp.float32),
    )(weight_q.T)
    return _sc_call(idxp, tbl.reshape(nq, EMBED_DIM), er)


# speedup vs baseline: 10.3773x; 10.3773x over previous
import jax
import jax.numpy as jnp
from jax import lax
from jax.experimental import pallas as pl
from jax.experimental.pallas import tpu as pltpu
from jax.experimental.pallas import tpu_sc as plsc

NUM_COLLISIONS = 4
EMBED_DIM = 64
BATCH = 16384
BAG = 20

NC = 2
NS = 16
NW = NC * NS
LANES = 16

BAGS_PER_W = BATCH // NW
CHUNK = 32
NCHUNK = BAGS_PER_W // CHUNK
ROWS_PER_CHUNK = CHUNK * BAG
IDX_GROUPS = ROWS_PER_CHUNK // 128
IDX_ROWS_W = BAGS_PER_W * BAG // 128
DSLICES = EMBED_DIM // LANES


REPACK_BLKC = 1024


def _repack_body(a_ref, b_ref, o_ref):
    o_ref[...] = jnp.concatenate(
        [a_ref[...].T, b_ref[...].T], axis=1)


def _er_body(idx_ref, wr_ref, o_ref):
    r = idx_ref[...] & 3
    wr = wr_ref[...]
    acc = jnp.zeros((idx_ref.shape[0], EMBED_DIM), jnp.float32)
    for k in range(NUM_COLLISIONS):
        cnt = jnp.sum((r == k).astype(jnp.float32), axis=1, keepdims=True)
        acc = acc + cnt * wr[k:k + 1, :]
    o_ref[...] = acc * (1.0 / (BAG * BAG))


def _sc_body(idx_hbm, wq_hbm, er_hbm, out_hbm,
             idxp_v, qflat, rows_a, rows_b, er_a, er_b, out_a, out_b,
             sem_a, sem_b, sem_oa, sem_ob):
    wid = lax.axis_index("s") * NC + lax.axis_index("c")
    bag_base = wid * BAGS_PER_W

    QT = BAGS_PER_W // 4
    for qt in range(4):
        pltpu.sync_copy(idx_hbm.at[pl.ds(bag_base + qt * QT, QT)], idxp_v)

        def compact_body(b, _, qt=qt):
            def to_row(v):
                q = jnp.right_shift(v, 2)
                r = q & 2047
                return (q - r) + 2 * (r & 1023) + jnp.right_shift(r, 10)
            lo = to_row(idxp_v[b, pl.ds(0, LANES)])
            hi = to_row(idxp_v[b, pl.ds(LANES, LANES)])
            off = (qt * QT + b) * BAG
            qflat[pl.ds(off, LANES)] = lo
            qflat[pl.ds(off + LANES, LANES)] = hi
            return 0
        lax.fori_loop(0, QT, compact_body, 0)

    def fire(c, rows_v, er_v, sem):
        for k in range(IDX_GROUPS):
            pltpu.async_copy(
                wq_hbm.at[qflat.at[pl.ds(c * ROWS_PER_CHUNK + k * 128, 128)]],
                rows_v.at[pl.ds(k * 128, 128)], sem)
        pltpu.async_copy(er_hbm.at[pl.ds(bag_base + c * CHUNK, CHUNK)],
                         er_v, sem)

    def wait_set(rows_v, er_v, sem):
        pltpu.make_async_copy(wq_hbm.at[pl.ds(0, ROWS_PER_CHUNK)],
                              rows_v, sem).wait()
        pltpu.make_async_copy(er_hbm.at[pl.ds(0, CHUNK)], er_v, sem).wait()

    def accum(c, rows_v, er_v, out_v, sem_o, guard):
        @pl.when(guard)
        def _():
            pltpu.make_async_copy(out_v, out_hbm.at[pl.ds(0, CHUNK)],
                                  sem_o).wait()

        def bag_body(b, _):
            rbase = b * BAG
            accs = [jnp.zeros((LANES,), jnp.float32) for _ in range(DSLICES)]
            for j in range(BAG):
                for s in range(DSLICES):
                    accs[s] = accs[s] + rows_v[rbase + j,
                                               pl.ds(s * LANES, LANES)]
            for s in range(DSLICES):
                out_v[b, pl.ds(s * LANES, LANES)] = (
                    accs[s] * er_v[b, pl.ds(s * LANES, LANES)])
            return 0

        lax.fori_loop(0, CHUNK, bag_body, 0)
        pltpu.async_copy(out_v, out_hbm.at[pl.ds(bag_base + c * CHUNK, CHUNK)],
                         sem_o)

    fire(0, rows_a, er_a, sem_a)

    def pair_body(p, _):
        c0 = 2 * p
        fire(c0 + 1, rows_b, er_b, sem_b)
        wait_set(rows_a, er_a, sem_a)
        accum(c0, rows_a, er_a, out_a, sem_oa, p > 0)

        @pl.when(p < NCHUNK // 2 - 1)
        def _():
            fire(c0 + 2, rows_a, er_a, sem_a)

        wait_set(rows_b, er_b, sem_b)
        accum(c0 + 1, rows_b, er_b, out_b, sem_ob, p > 0)
        return 0

    lax.fori_loop(0, NCHUNK // 2, pair_body, 0)
    pltpu.make_async_copy(out_a, out_hbm.at[pl.ds(0, CHUNK)], sem_oa).wait()
    pltpu.make_async_copy(out_b, out_hbm.at[pl.ds(0, CHUNK)], sem_ob).wait()


_sc_call = pl.kernel(
    _sc_body,
    out_type=jax.ShapeDtypeStruct((BATCH, EMBED_DIM), jnp.float32),
    mesh=plsc.VectorSubcoreMesh(core_axis_name="c", subcore_axis_name="s"),
    scratch_types=[
        pltpu.VMEM((BAGS_PER_W // 4, 128), jnp.int32),
        pltpu.VMEM((BAGS_PER_W * BAG + LANES, ), jnp.int32),
        pltpu.VMEM((ROWS_PER_CHUNK, EMBED_DIM), jnp.float32),
        pltpu.VMEM((ROWS_PER_CHUNK, EMBED_DIM), jnp.float32),
        pltpu.VMEM((CHUNK, EMBED_DIM), jnp.float32),
        pltpu.VMEM((CHUNK, EMBED_DIM), jnp.float32),
        pltpu.VMEM((CHUNK, EMBED_DIM), jnp.float32),
        pltpu.VMEM((CHUNK, EMBED_DIM), jnp.float32),
        pltpu.SemaphoreType.DMA,
        pltpu.SemaphoreType.DMA,
        pltpu.SemaphoreType.DMA,
        pltpu.SemaphoreType.DMA,
    ],
    compiler_params=pltpu.CompilerParams(use_tc_tiling_on_sc=False),
)


def kernel(input, weight_q, weight_r):
    idx = input.astype(jnp.int32)
    blk = 2048
    er = pl.pallas_call(
        _er_body,
        grid=(BATCH // blk,),
        in_specs=[
            pl.BlockSpec((blk, BAG), lambda i: (i, 0)),
            pl.BlockSpec((NUM_COLLISIONS, EMBED_DIM), lambda i: (0, 0)),
        ],
        out_specs=pl.BlockSpec((blk, EMBED_DIM), lambda i: (i, 0)),
        out_shape=jax.ShapeDtypeStruct((BATCH, EMBED_DIM), jnp.float32),
    )(idx, weight_r)
    idxp = jnp.pad(idx, ((0, 0), (0, 128 - BAG)))
    nq = weight_q.shape[0]
    nblk = pl.cdiv(nq, 2 * REPACK_BLKC)
    nlast = (nq - 1) // REPACK_BLKC
    wqt = weight_q.T
    tbl = pl.pallas_call(
        _repack_body,
        grid=(nblk,),
        in_specs=[
            pl.BlockSpec((EMBED_DIM, REPACK_BLKC),
                         lambda i: (0, jnp.minimum(2 * i, nlast))),
            pl.BlockSpec((EMBED_DIM, REPACK_BLKC),
                         lambda i: (0, jnp.minimum(2 * i + 1, nlast))),
        ],
        out_specs=pl.BlockSpec((REPACK_BLKC, 128), lambda i: (i, 0)),
        out_shape=jax.ShapeDtypeStruct((nblk * REPACK_BLKC, 128), jnp.float32),
    )(wqt, wqt)
    return _sc_call(idxp, tbl.reshape(nblk * REPACK_BLKC * 2, EMBED_DIM), er)


# speedup vs baseline: 13.1330x; 1.2656x over previous
import jax
import jax.numpy as jnp
from jax import lax
from jax.experimental import pallas as pl
from jax.experimental.pallas import tpu as pltpu
from jax.experimental.pallas import tpu_sc as plsc

NUM_COLLISIONS = 4
EMBED_DIM = 64
BATCH = 16384
BAG = 20

NC = 2
NS = 16
NW = NC * NS
LANES = 16

BAGS_PER_W = BATCH // NW
CHUNK = 32
NCHUNK = BAGS_PER_W // CHUNK
ROWS_PER_CHUNK = CHUNK * BAG
IDX_GROUPS = ROWS_PER_CHUNK // 128
IDX_ROWS_W = BAGS_PER_W * BAG // 128
DSLICES = EMBED_DIM // LANES


REPACK_BLKC = 4096


def _repack_body(a_ref, b_ref, o_ref):
    o_ref[...] = jnp.concatenate(
        [a_ref[...].T, b_ref[...].T], axis=1)


def _er_body(idx_ref, wr_ref, o_ref):
    r = idx_ref[...] & 3
    wr = wr_ref[...]
    acc = jnp.zeros((idx_ref.shape[0], EMBED_DIM), jnp.float32)
    for k in range(NUM_COLLISIONS):
        cnt = jnp.sum((r == k).astype(jnp.float32), axis=1, keepdims=True)
        acc = acc + cnt * wr[k:k + 1, :]
    o_ref[...] = acc * (1.0 / (BAG * BAG))


def _sc_body(idx_hbm, wq_hbm, er_hbm, out_hbm,
             idxp_v, qflat, rows_a, rows_b, er_a, er_b, out_a, out_b,
             sem_a, sem_b, sem_oa, sem_ob):
    wid = lax.axis_index("s") * NC + lax.axis_index("c")
    bag_base = wid * BAGS_PER_W

    QT = BAGS_PER_W // 4
    for qt in range(4):
        pltpu.sync_copy(idx_hbm.at[pl.ds(bag_base + qt * QT, QT)], idxp_v)

        def compact_body(b, _, qt=qt):
            def to_row(v):
                q = jnp.right_shift(v, 2)
                r = q & (2 * REPACK_BLKC - 1)
                return ((q - r) + 2 * (r & (REPACK_BLKC - 1))
                        + jnp.right_shift(r, REPACK_BLKC.bit_length() - 1))
            lo = to_row(idxp_v[b, pl.ds(0, LANES)])
            hi = to_row(idxp_v[b, pl.ds(LANES, LANES)])
            off = (qt * QT + b) * BAG
            qflat[pl.ds(off, LANES)] = lo
            qflat[pl.ds(off + LANES, LANES)] = hi
            return 0
        lax.fori_loop(0, QT, compact_body, 0)

    def fire(c, rows_v, er_v, sem):
        for k in range(IDX_GROUPS):
            pltpu.async_copy(
                wq_hbm.at[qflat.at[pl.ds(c * ROWS_PER_CHUNK + k * 128, 128)]],
                rows_v.at[pl.ds(k * 128, 128)], sem)
        pltpu.async_copy(er_hbm.at[pl.ds(bag_base + c * CHUNK, CHUNK)],
                         er_v, sem)

    def wait_set(rows_v, er_v, sem):
        pltpu.make_async_copy(wq_hbm.at[pl.ds(0, ROWS_PER_CHUNK)],
                              rows_v, sem).wait()
        pltpu.make_async_copy(er_hbm.at[pl.ds(0, CHUNK)], er_v, sem).wait()

    def accum(c, rows_v, er_v, out_v, sem_o, guard):
        @pl.when(guard)
        def _():
            pltpu.make_async_copy(out_v, out_hbm.at[pl.ds(0, CHUNK)],
                                  sem_o).wait()

        def bag_body(b, _):
            rbase = b * BAG
            accs = [jnp.zeros((LANES,), jnp.float32) for _ in range(DSLICES)]
            for j in range(BAG):
                for s in range(DSLICES):
                    accs[s] = accs[s] + rows_v[rbase + j,
                                               pl.ds(s * LANES, LANES)]
            for s in range(DSLICES):
                out_v[b, pl.ds(s * LANES, LANES)] = (
                    accs[s] * er_v[b, pl.ds(s * LANES, LANES)])
            return 0

        lax.fori_loop(0, CHUNK, bag_body, 0)
        pltpu.async_copy(out_v, out_hbm.at[pl.ds(bag_base + c * CHUNK, CHUNK)],
                         sem_o)

    fire(0, rows_a, er_a, sem_a)

    def pair_body(p, _):
        c0 = 2 * p
        fire(c0 + 1, rows_b, er_b, sem_b)
        wait_set(rows_a, er_a, sem_a)
        accum(c0, rows_a, er_a, out_a, sem_oa, p > 0)

        @pl.when(p < NCHUNK // 2 - 1)
        def _():
            fire(c0 + 2, rows_a, er_a, sem_a)

        wait_set(rows_b, er_b, sem_b)
        accum(c0 + 1, rows_b, er_b, out_b, sem_ob, p > 0)
        return 0

    lax.fori_loop(0, NCHUNK // 2, pair_body, 0)
    pltpu.make_async_copy(out_a, out_hbm.at[pl.ds(0, CHUNK)], sem_oa).wait()
    pltpu.make_async_copy(out_b, out_hbm.at[pl.ds(0, CHUNK)], sem_ob).wait()


_sc_call = pl.kernel(
    _sc_body,
    out_type=jax.ShapeDtypeStruct((BATCH, EMBED_DIM), jnp.float32),
    mesh=plsc.VectorSubcoreMesh(core_axis_name="c", subcore_axis_name="s"),
    scratch_types=[
        pltpu.VMEM((BAGS_PER_W // 4, 128), jnp.int32),
        pltpu.VMEM((BAGS_PER_W * BAG + LANES, ), jnp.int32),
        pltpu.VMEM((ROWS_PER_CHUNK, EMBED_DIM), jnp.float32),
        pltpu.VMEM((ROWS_PER_CHUNK, EMBED_DIM), jnp.float32),
        pltpu.VMEM((CHUNK, EMBED_DIM), jnp.float32),
        pltpu.VMEM((CHUNK, EMBED_DIM), jnp.float32),
        pltpu.VMEM((CHUNK, EMBED_DIM), jnp.float32),
        pltpu.VMEM((CHUNK, EMBED_DIM), jnp.float32),
        pltpu.SemaphoreType.DMA,
        pltpu.SemaphoreType.DMA,
        pltpu.SemaphoreType.DMA,
        pltpu.SemaphoreType.DMA,
    ],
    compiler_params=pltpu.CompilerParams(use_tc_tiling_on_sc=False),
)


def kernel(input, weight_q, weight_r):
    idx = input.astype(jnp.int32)
    blk = 2048
    er = pl.pallas_call(
        _er_body,
        grid=(BATCH // blk,),
        in_specs=[
            pl.BlockSpec((blk, BAG), lambda i: (i, 0)),
            pl.BlockSpec((NUM_COLLISIONS, EMBED_DIM), lambda i: (0, 0)),
        ],
        out_specs=pl.BlockSpec((blk, EMBED_DIM), lambda i: (i, 0)),
        out_shape=jax.ShapeDtypeStruct((BATCH, EMBED_DIM), jnp.float32),
    )(idx, weight_r)
    idxp = jnp.pad(idx, ((0, 0), (0, 128 - BAG)))
    nq = weight_q.shape[0]
    nblk = pl.cdiv(nq, 2 * REPACK_BLKC)
    nlast = (nq - 1) // REPACK_BLKC
    wqt = weight_q.T
    tbl = pl.pallas_call(
        _repack_body,
        grid=(nblk,),
        in_specs=[
            pl.BlockSpec((EMBED_DIM, REPACK_BLKC),
                         lambda i: (0, jnp.minimum(2 * i, nlast))),
            pl.BlockSpec((EMBED_DIM, REPACK_BLKC),
                         lambda i: (0, jnp.minimum(2 * i + 1, nlast))),
        ],
        out_specs=pl.BlockSpec((REPACK_BLKC, 128), lambda i: (i, 0)),
        out_shape=jax.ShapeDtypeStruct((nblk * REPACK_BLKC, 128), jnp.float32),
    )(wqt, wqt)
    return _sc_call(idxp, tbl.reshape(nblk * REPACK_BLKC * 2, EMBED_DIM), er)


# speedup vs baseline: 13.5855x; 1.0345x over previous
import jax
import jax.numpy as jnp
from jax import lax
from jax.experimental import pallas as pl
from jax.experimental.pallas import tpu as pltpu
from jax.experimental.pallas import tpu_sc as plsc

NUM_COLLISIONS = 4
EMBED_DIM = 64
BATCH = 16384
BAG = 20

NC = 2
NS = 16
NW = NC * NS
LANES = 16

BAGS_PER_W = BATCH // NW
CHUNK = 32
NCHUNK = BAGS_PER_W // CHUNK
ROWS_PER_CHUNK = CHUNK * BAG
IDX_GROUPS = ROWS_PER_CHUNK // 128
IDX_ROWS_W = BAGS_PER_W * BAG // 128
DSLICES = EMBED_DIM // LANES


REPACK_BLKC = 8192


def _repack_body(a_ref, b_ref, o_ref):
    o_ref[...] = jnp.concatenate(
        [a_ref[...].T, b_ref[...].T], axis=1)


def _er_body(idx_ref, wr_ref, o_ref):
    r = idx_ref[...] & 3
    wr = wr_ref[...]
    acc = jnp.zeros((idx_ref.shape[0], EMBED_DIM), jnp.float32)
    for k in range(NUM_COLLISIONS):
        cnt = jnp.sum((r == k).astype(jnp.float32), axis=1, keepdims=True)
        acc = acc + cnt * wr[k:k + 1, :]
    o_ref[...] = acc * (1.0 / (BAG * BAG))


def _sc_body(idx_hbm, wq_hbm, er_hbm, out_hbm,
             idxp_v, qflat, rows_a, rows_b, er_a, er_b, out_a, out_b,
             sem_a, sem_b, sem_oa, sem_ob):
    wid = lax.axis_index("s") * NC + lax.axis_index("c")
    bag_base = wid * BAGS_PER_W

    QT = BAGS_PER_W // 4
    for qt in range(4):
        pltpu.sync_copy(idx_hbm.at[pl.ds(bag_base + qt * QT, QT)], idxp_v)

        def compact_body(b, _, qt=qt):
            def to_row(v):
                q = jnp.right_shift(v, 2)
                r = q & (2 * REPACK_BLKC - 1)
                return ((q - r) + 2 * (r & (REPACK_BLKC - 1))
                        + jnp.right_shift(r, REPACK_BLKC.bit_length() - 1))
            lo = to_row(idxp_v[b, pl.ds(0, LANES)])
            hi = to_row(idxp_v[b, pl.ds(LANES, LANES)])
            off = (qt * QT + b) * BAG
            qflat[pl.ds(off, LANES)] = lo
            qflat[pl.ds(off + LANES, LANES)] = hi
            return 0
        lax.fori_loop(0, QT, compact_body, 0)

    def fire(c, rows_v, er_v, sem):
        for k in range(IDX_GROUPS):
            pltpu.async_copy(
                wq_hbm.at[qflat.at[pl.ds(c * ROWS_PER_CHUNK + k * 128, 128)]],
                rows_v.at[pl.ds(k * 128, 128)], sem)
        pltpu.async_copy(er_hbm.at[pl.ds(bag_base + c * CHUNK, CHUNK)],
                         er_v, sem)

    def wait_set(rows_v, er_v, sem):
        pltpu.make_async_copy(wq_hbm.at[pl.ds(0, ROWS_PER_CHUNK)],
                              rows_v, sem).wait()
        pltpu.make_async_copy(er_hbm.at[pl.ds(0, CHUNK)], er_v, sem).wait()

    def accum(c, rows_v, er_v, out_v, sem_o, guard):
        @pl.when(guard)
        def _():
            pltpu.make_async_copy(out_v, out_hbm.at[pl.ds(0, CHUNK)],
                                  sem_o).wait()

        def bag_body(b, _):
            rbase = b * BAG
            accs = [jnp.zeros((LANES,), jnp.float32) for _ in range(DSLICES)]
            for j in range(BAG):
                for s in range(DSLICES):
                    accs[s] = accs[s] + rows_v[rbase + j,
                                               pl.ds(s * LANES, LANES)]
            for s in range(DSLICES):
                out_v[b, pl.ds(s * LANES, LANES)] = (
                    accs[s] * er_v[b, pl.ds(s * LANES, LANES)])
            return 0

        lax.fori_loop(0, CHUNK, bag_body, 0)
        pltpu.async_copy(out_v, out_hbm.at[pl.ds(bag_base + c * CHUNK, CHUNK)],
                         sem_o)

    fire(0, rows_a, er_a, sem_a)

    def pair_body(p, _):
        c0 = 2 * p
        fire(c0 + 1, rows_b, er_b, sem_b)
        wait_set(rows_a, er_a, sem_a)
        accum(c0, rows_a, er_a, out_a, sem_oa, p > 0)

        @pl.when(p < NCHUNK // 2 - 1)
        def _():
            fire(c0 + 2, rows_a, er_a, sem_a)

        wait_set(rows_b, er_b, sem_b)
        accum(c0 + 1, rows_b, er_b, out_b, sem_ob, p > 0)
        return 0

    lax.fori_loop(0, NCHUNK // 2, pair_body, 0)
    pltpu.make_async_copy(out_a, out_hbm.at[pl.ds(0, CHUNK)], sem_oa).wait()
    pltpu.make_async_copy(out_b, out_hbm.at[pl.ds(0, CHUNK)], sem_ob).wait()


_sc_call = pl.kernel(
    _sc_body,
    out_type=jax.ShapeDtypeStruct((BATCH, EMBED_DIM), jnp.float32),
    mesh=plsc.VectorSubcoreMesh(core_axis_name="c", subcore_axis_name="s"),
    scratch_types=[
        pltpu.VMEM((BAGS_PER_W // 4, 128), jnp.int32),
        pltpu.VMEM((BAGS_PER_W * BAG + LANES, ), jnp.int32),
        pltpu.VMEM((ROWS_PER_CHUNK, EMBED_DIM), jnp.float32),
        pltpu.VMEM((ROWS_PER_CHUNK, EMBED_DIM), jnp.float32),
        pltpu.VMEM((CHUNK, EMBED_DIM), jnp.float32),
        pltpu.VMEM((CHUNK, EMBED_DIM), jnp.float32),
        pltpu.VMEM((CHUNK, EMBED_DIM), jnp.float32),
        pltpu.VMEM((CHUNK, EMBED_DIM), jnp.float32),
        pltpu.SemaphoreType.DMA,
        pltpu.SemaphoreType.DMA,
        pltpu.SemaphoreType.DMA,
        pltpu.SemaphoreType.DMA,
    ],
    compiler_params=pltpu.CompilerParams(use_tc_tiling_on_sc=False),
)


def kernel(input, weight_q, weight_r):
    idx = input.astype(jnp.int32)
    blk = 2048
    er = pl.pallas_call(
        _er_body,
        grid=(BATCH // blk,),
        in_specs=[
            pl.BlockSpec((blk, BAG), lambda i: (i, 0)),
            pl.BlockSpec((NUM_COLLISIONS, EMBED_DIM), lambda i: (0, 0)),
        ],
        out_specs=pl.BlockSpec((blk, EMBED_DIM), lambda i: (i, 0)),
        out_shape=jax.ShapeDtypeStruct((BATCH, EMBED_DIM), jnp.float32),
    )(idx, weight_r)
    idxp = jnp.pad(idx, ((0, 0), (0, 128 - BAG)))
    nq = weight_q.shape[0]
    nblk = pl.cdiv(nq, 2 * REPACK_BLKC)
    nlast = (nq - 1) // REPACK_BLKC
    wqt = weight_q.T
    tbl = pl.pallas_call(
        _repack_body,
        grid=(nblk,),
        in_specs=[
            pl.BlockSpec((EMBED_DIM, REPACK_BLKC),
                         lambda i: (0, jnp.minimum(2 * i, nlast))),
            pl.BlockSpec((EMBED_DIM, REPACK_BLKC),
                         lambda i: (0, jnp.minimum(2 * i + 1, nlast))),
        ],
        out_specs=pl.BlockSpec((REPACK_BLKC, 128), lambda i: (i, 0)),
        out_shape=jax.ShapeDtypeStruct((nblk * REPACK_BLKC, 128), jnp.float32),
    )(wqt, wqt)
    return _sc_call(idxp, tbl.reshape(nblk * REPACK_BLKC * 2, EMBED_DIM), er)


# speedup vs baseline: 13.9820x; 1.0292x over previous
import jax
import jax.numpy as jnp
from jax import lax
from jax.experimental import pallas as pl
from jax.experimental.pallas import tpu as pltpu
from jax.experimental.pallas import tpu_sc as plsc

NUM_COLLISIONS = 4
EMBED_DIM = 64
BATCH = 16384
BAG = 20

NC = 2
NS = 16
NW = NC * NS
LANES = 16

BAGS_PER_W = BATCH // NW
CHUNK = 32
NCHUNK = BAGS_PER_W // CHUNK
ROWS_PER_CHUNK = CHUNK * BAG
IDX_GROUPS = ROWS_PER_CHUNK // 128
IDX_ROWS_W = BAGS_PER_W * BAG // 128
DSLICES = EMBED_DIM // LANES


REPACK_BLKC = 8192


def _repack_body(a_ref, b_ref, o_ref):
    o_ref[...] = jnp.concatenate(
        [a_ref[...].T, b_ref[...].T], axis=1)


def _er_body(idxa_ref, idxb_ref, wr_ref, o_ref):
    wr = wr_ref[...]

    def half(idx_ref):
        r = idx_ref[...] & 3
        acc = jnp.zeros((idx_ref.shape[0], EMBED_DIM), jnp.float32)
        for k in range(NUM_COLLISIONS):
            cnt = jnp.sum((r == k).astype(jnp.float32), axis=1, keepdims=True)
            acc = acc + cnt * wr[k:k + 1, :]
        return acc * (1.0 / (BAG * BAG))

    o_ref[...] = jnp.concatenate([half(idxa_ref), half(idxb_ref)], axis=1)


def _sc_body(idx_hbm, wq_hbm, er_hbm, out_hbm,
             idxp_v, qflat, rows_a, rows_b, er_a, er_b, out_a, out_b,
             sem_a, sem_b, sem_oa, sem_ob):
    wid = lax.axis_index("s") * NC + lax.axis_index("c")
    bag_base = wid * BAGS_PER_W
    hoff = wid // (NW // 2)
    er_base = bag_base - hoff * (BATCH // 2)
    loff = hoff * EMBED_DIM

    QT = BAGS_PER_W // 4
    for qt in range(4):
        pltpu.sync_copy(idx_hbm.at[pl.ds(bag_base + qt * QT, QT)], idxp_v)

        def compact_body(b, _, qt=qt):
            def to_row(v):
                q = jnp.right_shift(v, 2)
                r = q & (2 * REPACK_BLKC - 1)
                return ((q - r) + 2 * (r & (REPACK_BLKC - 1))
                        + jnp.right_shift(r, REPACK_BLKC.bit_length() - 1))
            lo = to_row(idxp_v[b, pl.ds(0, LANES)])
            hi = to_row(idxp_v[b, pl.ds(LANES, LANES)])
            off = (qt * QT + b) * BAG
            qflat[pl.ds(off, LANES)] = lo
            qflat[pl.ds(off + LANES, LANES)] = hi
            return 0
        lax.fori_loop(0, QT, compact_body, 0)

    def fire(c, rows_v, er_v, sem):
        for k in range(IDX_GROUPS):
            pltpu.async_copy(
                wq_hbm.at[qflat.at[pl.ds(c * ROWS_PER_CHUNK + k * 128, 128)]],
                rows_v.at[pl.ds(k * 128, 128)], sem)
        pltpu.async_copy(er_hbm.at[pl.ds(er_base + c * CHUNK, CHUNK)],
                         er_v, sem)

    def wait_set(rows_v, er_v, sem):
        pltpu.make_async_copy(wq_hbm.at[pl.ds(0, ROWS_PER_CHUNK)],
                              rows_v, sem).wait()
        pltpu.make_async_copy(er_hbm.at[pl.ds(0, CHUNK)], er_v, sem).wait()

    def accum(c, rows_v, er_v, out_v, sem_o, guard):
        @pl.when(guard)
        def _():
            pltpu.make_async_copy(out_v, out_hbm.at[pl.ds(0, CHUNK // 2)],
                                  sem_o).wait()

        def bag_body(b, _):
            rbase = b * BAG
            accs = [jnp.zeros((LANES,), jnp.float32) for _ in range(DSLICES)]
            for j in range(BAG):
                for s in range(DSLICES):
                    accs[s] = accs[s] + rows_v[rbase + j,
                                               pl.ds(s * LANES, LANES)]
            orow = jnp.right_shift(b, 1)
            ocol = (b & 1) * EMBED_DIM
            for s in range(DSLICES):
                out_v[orow, pl.ds(ocol + s * LANES, LANES)] = (
                    accs[s] * er_v[b, pl.ds(loff + s * LANES, LANES)])
            return 0

        lax.fori_loop(0, CHUNK, bag_body, 0)
        pltpu.async_copy(
            out_v,
            out_hbm.at[pl.ds((bag_base + c * CHUNK) // 2, CHUNK // 2)],
            sem_o)

    fire(0, rows_a, er_a, sem_a)

    def pair_body(p, _):
        c0 = 2 * p
        fire(c0 + 1, rows_b, er_b, sem_b)
        wait_set(rows_a, er_a, sem_a)
        accum(c0, rows_a, er_a, out_a, sem_oa, p > 0)

        @pl.when(p < NCHUNK // 2 - 1)
        def _():
            fire(c0 + 2, rows_a, er_a, sem_a)

        wait_set(rows_b, er_b, sem_b)
        accum(c0 + 1, rows_b, er_b, out_b, sem_ob, p > 0)
        return 0

    lax.fori_loop(0, NCHUNK // 2, pair_body, 0)
    pltpu.make_async_copy(out_a, out_hbm.at[pl.ds(0, CHUNK // 2)],
                          sem_oa).wait()
    pltpu.make_async_copy(out_b, out_hbm.at[pl.ds(0, CHUNK // 2)],
                          sem_ob).wait()


_sc_call = pl.kernel(
    _sc_body,
    out_type=jax.ShapeDtypeStruct((BATCH // 2, 2 * EMBED_DIM), jnp.float32),
    mesh=plsc.VectorSubcoreMesh(core_axis_name="c", subcore_axis_name="s"),
    scratch_types=[
        pltpu.VMEM((BAGS_PER_W // 4, 128), jnp.int32),
        pltpu.VMEM((BAGS_PER_W * BAG + LANES, ), jnp.int32),
        pltpu.VMEM((ROWS_PER_CHUNK, EMBED_DIM), jnp.float32),
        pltpu.VMEM((ROWS_PER_CHUNK, EMBED_DIM), jnp.float32),
        pltpu.VMEM((CHUNK, 2 * EMBED_DIM), jnp.float32),
        pltpu.VMEM((CHUNK, 2 * EMBED_DIM), jnp.float32),
        pltpu.VMEM((CHUNK // 2, 2 * EMBED_DIM), jnp.float32),
        pltpu.VMEM((CHUNK // 2, 2 * EMBED_DIM), jnp.float32),
        pltpu.SemaphoreType.DMA,
        pltpu.SemaphoreType.DMA,
        pltpu.SemaphoreType.DMA,
        pltpu.SemaphoreType.DMA,
    ],
    compiler_params=pltpu.CompilerParams(use_tc_tiling_on_sc=False),
)


def kernel(input, weight_q, weight_r):
    idx = input.astype(jnp.int32)
    blk = 2048
    nhalf = (BATCH // 2) // blk
    er = pl.pallas_call(
        _er_body,
        grid=(nhalf,),
        in_specs=[
            pl.BlockSpec((blk, BAG), lambda i: (i, 0)),
            pl.BlockSpec((blk, BAG), lambda i: (i + nhalf, 0)),
            pl.BlockSpec((NUM_COLLISIONS, EMBED_DIM), lambda i: (0, 0)),
        ],
        out_specs=pl.BlockSpec((blk, 2 * EMBED_DIM), lambda i: (i, 0)),
        out_shape=jax.ShapeDtypeStruct((BATCH // 2, 2 * EMBED_DIM),
                                       jnp.float32),
    )(idx, idx, weight_r)
    idxp = jnp.pad(idx, ((0, 0), (0, 128 - BAG)))
    nq = weight_q.shape[0]
    nblk = pl.cdiv(nq, 2 * REPACK_BLKC)
    nlast = (nq - 1) // REPACK_BLKC
    wqt = weight_q.T
    tbl = pl.pallas_call(
        _repack_body,
        grid=(nblk,),
        in_specs=[
            pl.BlockSpec((EMBED_DIM, REPACK_BLKC),
                         lambda i: (0, jnp.minimum(2 * i, nlast))),
            pl.BlockSpec((EMBED_DIM, REPACK_BLKC),
                         lambda i: (0, jnp.minimum(2 * i + 1, nlast))),
        ],
        out_specs=pl.BlockSpec((REPACK_BLKC, 128), lambda i: (i, 0)),
        out_shape=jax.ShapeDtypeStruct((nblk * REPACK_BLKC, 128), jnp.float32),
    )(wqt, wqt)
    out = _sc_call(idxp, tbl.reshape(nblk * REPACK_BLKC * 2, EMBED_DIM), er)
    return out.reshape(BATCH, EMBED_DIM)


# speedup vs baseline: 14.1071x; 1.0089x over previous
import jax
import jax.numpy as jnp
from jax import lax
from jax.experimental import pallas as pl
from jax.experimental.pallas import tpu as pltpu
from jax.experimental.pallas import tpu_sc as plsc

NUM_COLLISIONS = 4
EMBED_DIM = 64
BATCH = 16384
BAG = 20

NC = 2
NS = 16
NW = NC * NS
LANES = 16

BAGS_PER_W = BATCH // NW
CHUNK = 32
NCHUNK = BAGS_PER_W // CHUNK
ROWS_PER_CHUNK = CHUNK * BAG
IDX_GROUPS = ROWS_PER_CHUNK // 128
IDX_ROWS_W = BAGS_PER_W * BAG // 128
DSLICES = EMBED_DIM // LANES


REPACK_BLKC = 16384


def _repack_body(a_ref, b_ref, o_ref):
    o_ref[...] = jnp.concatenate(
        [a_ref[...].T, b_ref[...].T], axis=1)


def _er_body(idxa_ref, idxb_ref, wr_ref, o_ref):
    wr = wr_ref[...]

    def half(idx_ref):
        r = idx_ref[...] & 3
        acc = jnp.zeros((idx_ref.shape[0], EMBED_DIM), jnp.float32)
        for k in range(NUM_COLLISIONS):
            cnt = jnp.sum((r == k).astype(jnp.float32), axis=1, keepdims=True)
            acc = acc + cnt * wr[k:k + 1, :]
        return acc * (1.0 / (BAG * BAG))

    o_ref[...] = jnp.concatenate([half(idxa_ref), half(idxb_ref)], axis=1)


def _sc_body(idx_hbm, wq_hbm, er_hbm, out_hbm,
             idxp_v, qflat, rows_a, rows_b, er_a, er_b, out_a, out_b,
             sem_a, sem_b, sem_oa, sem_ob):
    wid = lax.axis_index("s") * NC + lax.axis_index("c")
    bag_base = wid * BAGS_PER_W
    hoff = wid // (NW // 2)
    er_base = bag_base - hoff * (BATCH // 2)
    loff = hoff * EMBED_DIM

    QT = BAGS_PER_W // 4
    for qt in range(4):
        pltpu.sync_copy(idx_hbm.at[pl.ds(bag_base + qt * QT, QT)], idxp_v)

        def compact_body(b, _, qt=qt):
            def to_row(v):
                q = jnp.right_shift(v, 2)
                r = q & (2 * REPACK_BLKC - 1)
                return ((q - r) + 2 * (r & (REPACK_BLKC - 1))
                        + jnp.right_shift(r, REPACK_BLKC.bit_length() - 1))
            lo = to_row(idxp_v[b, pl.ds(0, LANES)])
            hi = to_row(idxp_v[b, pl.ds(LANES, LANES)])
            off = (qt * QT + b) * BAG
            qflat[pl.ds(off, LANES)] = lo
            qflat[pl.ds(off + LANES, LANES)] = hi
            return 0
        lax.fori_loop(0, QT, compact_body, 0)

    def fire(c, rows_v, er_v, sem):
        for k in range(IDX_GROUPS):
            pltpu.async_copy(
                wq_hbm.at[qflat.at[pl.ds(c * ROWS_PER_CHUNK + k * 128, 128)]],
                rows_v.at[pl.ds(k * 128, 128)], sem)
        pltpu.async_copy(er_hbm.at[pl.ds(er_base + c * CHUNK, CHUNK)],
                         er_v, sem)

    def wait_set(rows_v, er_v, sem):
        pltpu.make_async_copy(wq_hbm.at[pl.ds(0, ROWS_PER_CHUNK)],
                              rows_v, sem).wait()
        pltpu.make_async_copy(er_hbm.at[pl.ds(0, CHUNK)], er_v, sem).wait()

    def accum(c, rows_v, er_v, out_v, sem_o, guard):
        @pl.when(guard)
        def _():
            pltpu.make_async_copy(out_v, out_hbm.at[pl.ds(0, CHUNK // 2)],
                                  sem_o).wait()

        def bag_body(b, _):
            rbase = b * BAG
            accs = [jnp.zeros((LANES,), jnp.float32) for _ in range(DSLICES)]
            for j in range(BAG):
                for s in range(DSLICES):
                    accs[s] = accs[s] + rows_v[rbase + j,
                                               pl.ds(s * LANES, LANES)]
            orow = jnp.right_shift(b, 1)
            ocol = (b & 1) * EMBED_DIM
            for s in range(DSLICES):
                out_v[orow, pl.ds(ocol + s * LANES, LANES)] = (
                    accs[s] * er_v[b, pl.ds(loff + s * LANES, LANES)])
            return 0

        lax.fori_loop(0, CHUNK, bag_body, 0)
        pltpu.async_copy(
            out_v,
            out_hbm.at[pl.ds((bag_base + c * CHUNK) // 2, CHUNK // 2)],
            sem_o)

    fire(0, rows_a, er_a, sem_a)

    def pair_body(p, _):
        c0 = 2 * p
        fire(c0 + 1, rows_b, er_b, sem_b)
        wait_set(rows_a, er_a, sem_a)
        accum(c0, rows_a, er_a, out_a, sem_oa, p > 0)

        @pl.when(p < NCHUNK // 2 - 1)
        def _():
            fire(c0 + 2, rows_a, er_a, sem_a)

        wait_set(rows_b, er_b, sem_b)
        accum(c0 + 1, rows_b, er_b, out_b, sem_ob, p > 0)
        return 0

    lax.fori_loop(0, NCHUNK // 2, pair_body, 0)
    pltpu.make_async_copy(out_a, out_hbm.at[pl.ds(0, CHUNK // 2)],
                          sem_oa).wait()
    pltpu.make_async_copy(out_b, out_hbm.at[pl.ds(0, CHUNK // 2)],
                          sem_ob).wait()


_sc_call = pl.kernel(
    _sc_body,
    out_type=jax.ShapeDtypeStruct((BATCH // 2, 2 * EMBED_DIM), jnp.float32),
    mesh=plsc.VectorSubcoreMesh(core_axis_name="c", subcore_axis_name="s"),
    scratch_types=[
        pltpu.VMEM((BAGS_PER_W // 4, 128), jnp.int32),
        pltpu.VMEM((BAGS_PER_W * BAG + LANES, ), jnp.int32),
        pltpu.VMEM((ROWS_PER_CHUNK, EMBED_DIM), jnp.float32),
        pltpu.VMEM((ROWS_PER_CHUNK, EMBED_DIM), jnp.float32),
        pltpu.VMEM((CHUNK, 2 * EMBED_DIM), jnp.float32),
        pltpu.VMEM((CHUNK, 2 * EMBED_DIM), jnp.float32),
        pltpu.VMEM((CHUNK // 2, 2 * EMBED_DIM), jnp.float32),
        pltpu.VMEM((CHUNK // 2, 2 * EMBED_DIM), jnp.float32),
        pltpu.SemaphoreType.DMA,
        pltpu.SemaphoreType.DMA,
        pltpu.SemaphoreType.DMA,
        pltpu.SemaphoreType.DMA,
    ],
    compiler_params=pltpu.CompilerParams(use_tc_tiling_on_sc=False),
)


def kernel(input, weight_q, weight_r):
    idx = input.astype(jnp.int32)
    blk = 2048
    nhalf = (BATCH // 2) // blk
    er = pl.pallas_call(
        _er_body,
        grid=(nhalf,),
        in_specs=[
            pl.BlockSpec((blk, BAG), lambda i: (i, 0)),
            pl.BlockSpec((blk, BAG), lambda i: (i + nhalf, 0)),
            pl.BlockSpec((NUM_COLLISIONS, EMBED_DIM), lambda i: (0, 0)),
        ],
        out_specs=pl.BlockSpec((blk, 2 * EMBED_DIM), lambda i: (i, 0)),
        out_shape=jax.ShapeDtypeStruct((BATCH // 2, 2 * EMBED_DIM),
                                       jnp.float32),
    )(idx, idx, weight_r)
    idxp = jnp.pad(idx, ((0, 0), (0, 128 - BAG)))
    nq = weight_q.shape[0]
    nblk = pl.cdiv(nq, 2 * REPACK_BLKC)
    nlast = (nq - 1) // REPACK_BLKC
    wqt = weight_q.T
    tbl = pl.pallas_call(
        _repack_body,
        grid=(nblk,),
        in_specs=[
            pl.BlockSpec((EMBED_DIM, REPACK_BLKC),
                         lambda i: (0, jnp.minimum(2 * i, nlast))),
            pl.BlockSpec((EMBED_DIM, REPACK_BLKC),
                         lambda i: (0, jnp.minimum(2 * i + 1, nlast))),
        ],
        out_specs=pl.BlockSpec((REPACK_BLKC, 128), lambda i: (i, 0)),
        out_shape=jax.ShapeDtypeStruct((nblk * REPACK_BLKC, 128), jnp.float32),
    )(wqt, wqt)
    out = _sc_call(idxp, tbl.reshape(nblk * REPACK_BLKC * 2, EMBED_DIM), er)
    return out.reshape(BATCH, EMBED_DIM)
